# Initial kernel scaffold; baseline (speedup 1.0000x reference)
import jax
import jax.numpy as jnp
from jax.experimental import pallas as pl


def kernel(tokens, builtin_table, variable_table):
    raise NotImplementedError("write your pallas kernel here")


# speedup vs baseline: 1.1298x; 1.1298x over previous
import functools

import jax
import jax.numpy as jnp
from jax import lax
from jax.experimental import pallas as pl
from jax.experimental.pallas import tpu as pltpu
from jax.experimental.pallas import tpu_sc as plsc

_EMBED = 179
_OUT_D = _EMBED + 1
_TBL_ROWS = 128
_ZERO_ROW = _TBL_ROWS - 1
_PAIR_W = 2 * _OUT_D + 8

_NC, _NS, _L = 2, 16, 16
_NW = _NC * _NS

_CHUNK = 512
_PAIRS = _CHUNK // 2
_GSUB = 128


def _sc_body(n_tok, tokens_hbm, table2_hbm, out_hbm,
             tok_v, eff_v, pair_v, last_v, raw_v, gsem):
    wid = lax.axis_index("s") * _NC + lax.axis_index("c")
    per_w = n_tok // _NW
    tile_base = wid * per_w
    n_chunks = per_w // _CHUNK

    iota = lax.iota(jnp.int32, _L)
    col_a = jnp.full((_L,), _OUT_D - 1, jnp.int32)
    col_b = jnp.full((_L,), 2 * _OUT_D - 1, jnp.int32)

    def chunk_step(g, carry):
        base = tile_base + g * _CHUNK
        pltpu.sync_copy(tokens_hbm.at[pl.ds(base * 2, _CHUNK * 2)], tok_v)

        for k in range(_CHUNK // _L):
            rows16 = iota + (k * _L)
            flat16 = rows16 * 2
            i16 = plsc.load_gather(tok_v, [flat16])
            t16 = plsc.load_gather(tok_v, [flat16 + 1])
            is_b = t16 == 0
            eff16 = jnp.where(is_b, i16, _ZERO_ROW)
            last16 = jnp.where(is_b, jnp.float32(-1.0),
                               i16.astype(jnp.float32))
            eff_v[pl.ds(k * _L, _L)] = eff16
            last_v[pl.ds(k * _L, _L)] = last16

        for k in range(_PAIRS // _L):
            p2 = (iota + (k * _L)) * 2
            a16 = plsc.load_gather(eff_v, [p2])
            b16 = plsc.load_gather(eff_v, [p2 + 1])
            pair_v[k // (_GSUB // _L), pl.ds((k % (_GSUB // _L)) * _L, _L)] = (
                a16 * _TBL_ROWS + b16)

        copies = []
        for j in range(_PAIRS // _GSUB):
            copies.append(pltpu.async_copy(
                table2_hbm.at[pair_v.at[j]],
                raw_v.at[pl.ds(j * _GSUB, _GSUB)], gsem))
        for c in copies:
            c.wait()

        for k in range(_PAIRS // _L):
            prow16 = iota + (k * _L)
            tok2 = prow16 * 2
            la = plsc.load_gather(last_v, [tok2])
            lb = plsc.load_gather(last_v, [tok2 + 1])
            plsc.store_scatter(raw_v, [prow16, col_a], la)
            plsc.store_scatter(raw_v, [prow16, col_b], lb)

        pltpu.sync_copy(raw_v.at[:, pl.ds(0, 2 * _OUT_D)],
                        out_hbm.at[pl.ds(base // 2, _PAIRS)])
        return carry

    lax.fori_loop(0, n_chunks, chunk_step, 0)


@functools.partial(jax.jit, static_argnames=("n_tok",))
def _run(tokens, table2, n_tok):
    mesh = plsc.VectorSubcoreMesh(core_axis_name="c", subcore_axis_name="s")
    out = pl.kernel(
        functools.partial(_sc_body, n_tok),
        out_type=jax.ShapeDtypeStruct((n_tok // 2, 2 * _OUT_D), jnp.float32),
        mesh=mesh,
        compiler_params=pltpu.CompilerParams(
            needs_layout_passes=False, use_tc_tiling_on_sc=False),
        scratch_types=[
            pltpu.VMEM((_CHUNK * 2,), jnp.int32),
            pltpu.VMEM((_CHUNK,), jnp.int32),
            pltpu.VMEM((_PAIRS // _GSUB, _GSUB), jnp.int32),
            pltpu.VMEM((_CHUNK,), jnp.float32),
            pltpu.VMEM((_PAIRS, _PAIR_W), jnp.float32),
            pltpu.SemaphoreType.DMA,
        ],
    )(tokens.reshape(n_tok * 2), table2)
    return out.reshape(n_tok, _OUT_D)


def kernel(tokens, builtin_table, variable_table):
    del variable_table
    n_tok = tokens.shape[0]
    tbl = jnp.zeros((_TBL_ROWS, _OUT_D), jnp.float32)
    tbl = tbl.at[: builtin_table.shape[0], :_EMBED].set(builtin_table)
    tbl = tbl.at[: builtin_table.shape[0], _EMBED].set(-1.0)
    t2 = jnp.concatenate(
        [
            jnp.broadcast_to(tbl[:, None, :], (_TBL_ROWS, _TBL_ROWS, _OUT_D)),
            jnp.broadcast_to(tbl[None, :, :], (_TBL_ROWS, _TBL_ROWS, _OUT_D)),
            jnp.zeros((_TBL_ROWS, _TBL_ROWS, _PAIR_W - 2 * _OUT_D), jnp.float32),
        ],
        axis=-1,
    ).reshape(_TBL_ROWS * _TBL_ROWS, _PAIR_W)
    return _run(tokens, t2, n_tok)


# speedup vs baseline: 1.1313x; 1.0013x over previous
import functools

import jax
import jax.numpy as jnp
from jax import lax
from jax.experimental import pallas as pl
from jax.experimental.pallas import tpu as pltpu
from jax.experimental.pallas import tpu_sc as plsc

_EMBED = 179
_OUT_D = _EMBED + 1
_TBL_ROWS = 128
_ZERO_ROW = _TBL_ROWS - 1
_PAIR_W = 2 * _OUT_D + 8

_NC, _NS, _L = 2, 16, 16
_NW = _NC * _NS

_CHUNK = 256
_PAIRS = _CHUNK // 2


def _sc_body(n_tok, tokens_hbm, table2_hbm, out_hbm,
             tok0, tok1, eff0, eff1, pair0, pair1, last0, last1,
             raw0, raw1, tsem, gs0, gs1, os0, os1):
    wid = lax.axis_index("s") * _NC + lax.axis_index("c")
    per_w = n_tok // _NW
    tile_base = wid * per_w
    n_chunks = per_w // _CHUNK
    n2 = n_chunks // 2

    iota = lax.iota(jnp.int32, _L)
    col_a = jnp.full((_L,), _OUT_D - 1, jnp.int32)
    col_b = jnp.full((_L,), 2 * _OUT_D - 1, jnp.int32)

    def tok_src(g):
        return tokens_hbm.at[pl.ds((tile_base + g * _CHUNK) * 2, _CHUNK * 2)]

    def out_dst(g):
        return out_hbm.at[pl.ds((tile_base + g * _CHUNK) // 2, _PAIRS)]

    def parse(tok_v, eff_v, pair_v, last_v):
        for k in range(_CHUNK // _L):
            flat16 = (iota + (k * _L)) * 2
            i16 = plsc.load_gather(tok_v, [flat16])
            t16 = plsc.load_gather(tok_v, [flat16 + 1])
            is_b = t16 == 0
            eff_v[pl.ds(k * _L, _L)] = jnp.where(is_b, i16, _ZERO_ROW)
            last_v[pl.ds(k * _L, _L)] = jnp.where(
                is_b, jnp.float32(-1.0), i16.astype(jnp.float32))
        for k in range(_PAIRS // _L):
            p2 = (iota + (k * _L)) * 2
            a16 = plsc.load_gather(eff_v, [p2])
            b16 = plsc.load_gather(eff_v, [p2 + 1])
            pair_v[0, pl.ds(k * _L, _L)] = a16 * _TBL_ROWS + b16

    def patch(raw_v, last_v):
        for k in range(_PAIRS // _L):
            prow16 = iota + (k * _L)
            tok2 = prow16 * 2
            la = plsc.load_gather(last_v, [tok2])
            lb = plsc.load_gather(last_v, [tok2 + 1])
            plsc.store_scatter(raw_v, [prow16, col_a], la)
            plsc.store_scatter(raw_v, [prow16, col_b], lb)

    def gather_start(pair_v, raw_v, sem):
        pltpu.async_copy(table2_hbm.at[pair_v.at[0]], raw_v, sem)

    def gather_wait(pair_v, raw_v, sem):
        pltpu.make_async_copy(table2_hbm.at[pair_v.at[0]], raw_v, sem).wait()

    def out_start(g, raw_v, sem):
        pltpu.async_copy(raw_v.at[:, pl.ds(0, 2 * _OUT_D)], out_dst(g), sem)

    def out_wait(g, raw_v, sem):
        pltpu.make_async_copy(
            raw_v.at[:, pl.ds(0, 2 * _OUT_D)], out_dst(g), sem).wait()

    pltpu.sync_copy(tok_src(0), tok0)
    parse(tok0, eff0, pair0, last0)
    gather_start(pair0, raw0, gs0)

    def body(gg, carry):
        g = 2 * gg
        tok_h = pltpu.async_copy(tok_src(g + 1), tok1, tsem)
        gather_wait(pair0, raw0, gs0)
        patch(raw0, last0)
        out_start(g, raw0, os0)
        tok_h.wait()
        parse(tok1, eff1, pair1, last1)

        @pl.when(gg > 0)
        def _():
            out_wait(g - 1, raw1, os1)
        gather_start(pair1, raw1, gs1)

        gather_wait(pair1, raw1, gs1)
        patch(raw1, last1)
        out_start(g + 1, raw1, os1)

        @pl.when(gg < n2 - 1)
        def _():
            pltpu.sync_copy(tok_src(g + 2), tok0)
            parse(tok0, eff0, pair0, last0)
            out_wait(g, raw0, os0)
            gather_start(pair0, raw0, gs0)

        return carry

    lax.fori_loop(0, n2, body, 0)

    out_wait(n_chunks - 2, raw0, os0)
    out_wait(n_chunks - 1, raw1, os1)


@functools.partial(jax.jit, static_argnames=("n_tok",))
def _run(tokens, table2, n_tok):
    mesh = plsc.VectorSubcoreMesh(core_axis_name="c", subcore_axis_name="s")
    out = pl.kernel(
        functools.partial(_sc_body, n_tok),
        out_type=jax.ShapeDtypeStruct((n_tok // 2, 2 * _OUT_D), jnp.float32),
        mesh=mesh,
        compiler_params=pltpu.CompilerParams(
            needs_layout_passes=False, use_tc_tiling_on_sc=False),
        scratch_types=[
            pltpu.VMEM((_CHUNK * 2,), jnp.int32),
            pltpu.VMEM((_CHUNK * 2,), jnp.int32),
            pltpu.VMEM((_CHUNK,), jnp.int32),
            pltpu.VMEM((_CHUNK,), jnp.int32),
            pltpu.VMEM((1, _PAIRS), jnp.int32),
            pltpu.VMEM((1, _PAIRS), jnp.int32),
            pltpu.VMEM((_CHUNK,), jnp.float32),
            pltpu.VMEM((_CHUNK,), jnp.float32),
            pltpu.VMEM((_PAIRS, _PAIR_W), jnp.float32),
            pltpu.VMEM((_PAIRS, _PAIR_W), jnp.float32),
            pltpu.SemaphoreType.DMA,
            pltpu.SemaphoreType.DMA,
            pltpu.SemaphoreType.DMA,
            pltpu.SemaphoreType.DMA,
            pltpu.SemaphoreType.DMA,
        ],
    )(tokens.reshape(n_tok * 2), table2)
    return out.reshape(n_tok, _OUT_D)


def kernel(tokens, builtin_table, variable_table):
    del variable_table
    n_tok = tokens.shape[0]
    tbl = jnp.zeros((_TBL_ROWS, _OUT_D), jnp.float32)
    tbl = tbl.at[: builtin_table.shape[0], :_EMBED].set(builtin_table)
    tbl = tbl.at[: builtin_table.shape[0], _EMBED].set(-1.0)
    t2 = jnp.concatenate(
        [
            jnp.broadcast_to(tbl[:, None, :], (_TBL_ROWS, _TBL_ROWS, _OUT_D)),
            jnp.broadcast_to(tbl[None, :, :], (_TBL_ROWS, _TBL_ROWS, _OUT_D)),
            jnp.zeros((_TBL_ROWS, _TBL_ROWS, _PAIR_W - 2 * _OUT_D), jnp.float32),
        ],
        axis=-1,
    ).reshape(_TBL_ROWS * _TBL_ROWS, _PAIR_W)
    return _run(tokens, t2, n_tok)


# speedup vs baseline: 3.8009x; 3.3599x over previous
import functools

import jax
import jax.numpy as jnp
from jax import lax
from jax.experimental import pallas as pl
from jax.experimental.pallas import tpu as pltpu
from jax.experimental.pallas import tpu_sc as plsc

_EMBED = 179
_OUT_D = _EMBED + 1
_TBL_ROWS = 128
_ZERO_ROW = _TBL_ROWS - 1
_TBL_FLAT = _TBL_ROWS * _OUT_D

_NC, _NS, _L = 2, 16, 16
_NW = _NC * _NS

_CHUNK = 256


def _sc_body(n_tok, idx_hbm, typ_hbm, tbl_hbm, out_hbm,
             idx0, idx1, typ0, typ1, effb_v, last_v, tbl_v,
             raw0, raw1, ts0, ts1, os0, os1):
    wid = lax.axis_index("s") * _NC + lax.axis_index("c")
    per_w = n_tok // _NW
    tile_base = wid * per_w
    n_chunks = per_w // _CHUNK
    n2 = n_chunks // 2

    iota = lax.iota(jnp.int32, _L)

    def idx_src(g):
        return idx_hbm.at[pl.ds(tile_base + g * _CHUNK, _CHUNK)]

    def typ_src(g):
        return typ_hbm.at[pl.ds(tile_base + g * _CHUNK, _CHUNK)]

    def out_dst(g):
        return out_hbm.at[pl.ds((tile_base + g * _CHUNK) * _OUT_D,
                                _CHUNK * _OUT_D)]

    def parse(idx_v, typ_v):
        for k in range(_CHUNK // _L):
            i16 = idx_v[pl.ds(k * _L, _L)]
            t16 = typ_v[pl.ds(k * _L, _L)]
            is_b = t16 == 0
            eff16 = jnp.where(is_b, i16, _ZERO_ROW)
            effb_v[pl.ds(k * _L, _L)] = eff16 * _OUT_D
            last_v[pl.ds(k * _L, _L)] = jnp.where(
                is_b, jnp.float32(-1.0), i16.astype(jnp.float32))

    def distribute(raw_v):
        def group(k, carry):
            effb16 = effb_v[pl.ds(k * _L, _L)]
            rowf16 = iota * _OUT_D + k * (_L * _OUT_D)
            for c in range(_EMBED):
                a = plsc.load_gather(tbl_v, [effb16 + c])
                plsc.store_scatter(raw_v, [rowf16 + c], a)
            plsc.store_scatter(raw_v, [rowf16 + _EMBED],
                               last_v[pl.ds(k * _L, _L)])
            return carry

        lax.fori_loop(0, _CHUNK // _L, group, 0)

    def tok_start(g, idx_v, typ_v, sem):
        pltpu.async_copy(idx_src(g), idx_v, sem)
        pltpu.async_copy(typ_src(g), typ_v, sem)

    def tok_wait(g, idx_v, typ_v, sem):
        pltpu.make_async_copy(idx_src(g), idx_v, sem).wait()
        pltpu.make_async_copy(typ_src(g), typ_v, sem).wait()

    def out_start(g, raw_v, sem):
        pltpu.async_copy(raw_v, out_dst(g), sem)

    def out_wait(g, raw_v, sem):
        pltpu.make_async_copy(raw_v, out_dst(g), sem).wait()

    pltpu.sync_copy(tbl_hbm, tbl_v)
    pltpu.sync_copy(idx_src(0), idx0)
    pltpu.sync_copy(typ_src(0), typ0)

    def body(gg, carry):
        g = 2 * gg
        tok_start(g + 1, idx1, typ1, ts1)
        parse(idx0, typ0)

        @pl.when(gg > 0)
        def _():
            out_wait(g - 2, raw0, os0)
        distribute(raw0)
        out_start(g, raw0, os0)

        @pl.when(gg < n2 - 1)
        def _():
            tok_start(g + 2, idx0, typ0, ts0)
        tok_wait(g + 1, idx1, typ1, ts1)
        parse(idx1, typ1)

        @pl.when(gg > 0)
        def _():
            out_wait(g - 1, raw1, os1)
        distribute(raw1)
        out_start(g + 1, raw1, os1)

        @pl.when(gg < n2 - 1)
        def _():
            tok_wait(g + 2, idx0, typ0, ts0)

        return carry

    lax.fori_loop(0, n2, body, 0)

    out_wait(n_chunks - 2, raw0, os0)
    out_wait(n_chunks - 1, raw1, os1)


@functools.partial(jax.jit, static_argnames=("n_tok",))
def _run(idx, typ, tbl_flat, n_tok):
    mesh = plsc.VectorSubcoreMesh(core_axis_name="c", subcore_axis_name="s")
    out = pl.kernel(
        functools.partial(_sc_body, n_tok),
        out_type=jax.ShapeDtypeStruct((n_tok * _OUT_D,), jnp.float32),
        mesh=mesh,
        compiler_params=pltpu.CompilerParams(
            needs_layout_passes=False, use_tc_tiling_on_sc=False),
        scratch_types=[
            pltpu.VMEM((_CHUNK,), jnp.int32),
            pltpu.VMEM((_CHUNK,), jnp.int32),
            pltpu.VMEM((_CHUNK,), jnp.int32),
            pltpu.VMEM((_CHUNK,), jnp.int32),
            pltpu.VMEM((_CHUNK,), jnp.int32),
            pltpu.VMEM((_CHUNK,), jnp.float32),
            pltpu.VMEM((_TBL_FLAT,), jnp.float32),
            pltpu.VMEM((_CHUNK * _OUT_D,), jnp.float32),
            pltpu.VMEM((_CHUNK * _OUT_D,), jnp.float32),
            pltpu.SemaphoreType.DMA,
            pltpu.SemaphoreType.DMA,
            pltpu.SemaphoreType.DMA,
            pltpu.SemaphoreType.DMA,
        ],
    )(idx, typ, tbl_flat)
    return out.reshape(n_tok, _OUT_D)


def kernel(tokens, builtin_table, variable_table):
    del variable_table
    n_tok = tokens.shape[0]
    tbl = jnp.zeros((_TBL_ROWS, _OUT_D), jnp.float32)
    tbl = tbl.at[: builtin_table.shape[0], :_EMBED].set(builtin_table)
    tbl = tbl.at[: builtin_table.shape[0], _EMBED].set(-1.0)
    return _run(tokens[:, 0], tokens[:, 1], tbl.reshape(_TBL_FLAT), n_tok)
